# Initial kernel scaffold; baseline (speedup 1.0000x reference)
import jax
import jax.numpy as jnp
from jax.experimental import pallas as pl


def kernel(x, batch):
    raise NotImplementedError("write your pallas kernel here")


# speedup vs baseline: 6.1507x; 6.1507x over previous
import functools

import jax
import jax.numpy as jnp
from jax import lax
from jax.experimental import pallas as pl
from jax.experimental.pallas import tpu as pltpu
from jax.experimental.pallas import tpu_sc as plsc

NUM_SEGMENTS = 64
N_ROWS = 100000
D = 512
NC = 2
NS = 16
L = 16
NW = NC * NS
SEGS_PER_W = NUM_SEGMENTS // NW
NVEC = D // L
BLK = 64
STARTS_PAD = 88


def _seg_max_sc(x, starts):
    mesh = plsc.VectorSubcoreMesh(
        core_axis_name="c", subcore_axis_name="s",
        num_cores=NC, num_subcores=NS)

    @functools.partial(
        pl.kernel,
        out_type=jax.ShapeDtypeStruct((NUM_SEGMENTS * D,), jnp.float32),
        mesh=mesh,
        scratch_types=[
            pltpu.VMEM((STARTS_PAD,), jnp.int32),
            pltpu.VMEM((BLK, D), jnp.float32),
            pltpu.VMEM((BLK, D), jnp.float32),
            pltpu.VMEM((SEGS_PER_W * D,), jnp.float32),
            pltpu.SemaphoreType.DMA,
            pltpu.SemaphoreType.DMA,
            pltpu.SemaphoreType.DMA,
        ],
    )
    def k(x_hbm, starts_hbm, out_hbm, starts_v, buf0, buf1, res_v,
          sem0, sem1, sem_out):
        wid = lax.axis_index("s") * NC + lax.axis_index("c")
        pltpu.sync_copy(starts_hbm, starts_v)
        bufs = (buf0, buf1)
        sems = (sem0, sem1)

        for si in range(SEGS_PER_W):
            seg = wid * SEGS_PER_W + si
            bounds = starts_v[pl.ds(seg, L)]
            row_lo = bounds[0]
            row_hi = bounds[1]
            aligned_lo = (row_lo // 8) * 8
            nblk = (row_hi - aligned_lo + BLK - 1) // BLK

            def blk_base(i, aligned_lo=aligned_lo):
                return jnp.minimum(aligned_lo + i * BLK, N_ROWS - BLK)

            def start_dma(i, b):
                pltpu.async_copy(
                    x_hbm.at[pl.ds(blk_base(i), BLK)], bufs[b], sems[b])

            def wait_dma(b):
                pltpu.make_async_copy(
                    x_hbm.at[pl.ds(0, BLK)], bufs[b], sems[b]).wait()

            @pl.when(nblk > 0)
            def _():
                start_dma(0, 0)

            def process(i, b, acc, row_lo=row_lo, row_hi=row_hi):
                base = blk_base(i)
                lo_r = jnp.maximum(row_lo - base, 0)
                hi_r = jnp.minimum(row_hi - base, BLK)
                buf = bufs[b]

                def row_body(r, acc):
                    return tuple(
                        jnp.maximum(acc[j], buf[r, pl.ds(j * L, L)])
                        for j in range(NVEC))

                return lax.fori_loop(lo_r, hi_r, row_body, acc)

            def pair_body(p, acc, nblk=nblk):
                i0 = 2 * p
                i1 = i0 + 1

                @pl.when(i1 < nblk)
                def _():
                    start_dma(i1, 1)

                wait_dma(0)
                acc = process(i0, 0, acc)

                @pl.when(i1 + 1 < nblk)
                def _():
                    start_dma(i1 + 1, 0)

                @pl.when(i1 < nblk)
                def _():
                    wait_dma(1)

                acc = process(i1, 1, acc)
                return acc

            neg_inf = jnp.full((L,), -jnp.inf, dtype=jnp.float32)
            acc0 = tuple(neg_inf for _ in range(NVEC))
            npairs = (nblk + 1) // 2
            acc = lax.fori_loop(0, npairs, pair_body, acc0)

            for j in range(NVEC):
                res_v[pl.ds(si * D + j * L, L)] = acc[j]

        pltpu.async_copy(
            res_v, out_hbm.at[pl.ds(wid * SEGS_PER_W * D, SEGS_PER_W * D)],
            sem_out).wait()

    return k(x, starts)


def kernel(x, batch):
    seg_ids = jnp.arange(NUM_SEGMENTS + 1, dtype=batch.dtype)
    starts = jnp.searchsorted(batch, seg_ids, side="left").astype(jnp.int32)
    starts = jnp.pad(starts, (0, STARTS_PAD - NUM_SEGMENTS - 1))
    return _seg_max_sc(x, starts).reshape(NUM_SEGMENTS, D)


# speedup vs baseline: 7.6452x; 1.2430x over previous
import functools

import jax
import jax.numpy as jnp
from jax import lax
from jax.experimental import pallas as pl
from jax.experimental.pallas import tpu as pltpu
from jax.experimental.pallas import tpu_sc as plsc

NUM_SEGMENTS = 64
N_ROWS = 100000
D = 512
NC = 2
NS = 16
L = 16
NW = NC * NS
SEGS_PER_W = NUM_SEGMENTS // NW
NVEC = D // L
BLK = 64
STARTS_PAD = 88


def _seg_max_sc(x, starts):
    mesh = plsc.VectorSubcoreMesh(
        core_axis_name="c", subcore_axis_name="s",
        num_cores=NC, num_subcores=NS)

    @functools.partial(
        pl.kernel,
        out_type=jax.ShapeDtypeStruct((NUM_SEGMENTS * D,), jnp.float32),
        mesh=mesh,
        scratch_types=[
            pltpu.VMEM((STARTS_PAD,), jnp.int32),
            pltpu.VMEM((BLK, D), jnp.float32),
            pltpu.VMEM((BLK, D), jnp.float32),
            pltpu.VMEM((SEGS_PER_W * D,), jnp.float32),
            pltpu.SemaphoreType.DMA,
            pltpu.SemaphoreType.DMA,
            pltpu.SemaphoreType.DMA,
        ],
    )
    def k(x_hbm, starts_hbm, out_hbm, starts_v, buf0, buf1, res_v,
          sem0, sem1, sem_out):
        wid = lax.axis_index("s") * NC + lax.axis_index("c")
        pltpu.sync_copy(starts_hbm, starts_v)
        bufs = (buf0, buf1)
        sems = (sem0, sem1)

        for si in range(SEGS_PER_W):
            seg = wid * SEGS_PER_W + si
            bounds = starts_v[pl.ds(seg, L)]
            row_lo = bounds[0]
            row_hi = bounds[1]
            aligned_lo = (row_lo // 8) * 8
            nblk = (row_hi - aligned_lo + BLK - 1) // BLK

            def blk_base(i, aligned_lo=aligned_lo):
                return jnp.minimum(aligned_lo + i * BLK, N_ROWS - BLK)

            def start_dma(i, b):
                pltpu.async_copy(
                    x_hbm.at[pl.ds(blk_base(i), BLK)], bufs[b], sems[b])

            def wait_dma(b):
                pltpu.make_async_copy(
                    x_hbm.at[pl.ds(0, BLK)], bufs[b], sems[b]).wait()

            @pl.when(nblk > 0)
            def _():
                start_dma(0, 0)

            def process(i, b, acc, row_lo=row_lo, row_hi=row_hi):
                base = blk_base(i)
                lo_r = jnp.maximum(row_lo - base, 0)
                hi_r = jnp.minimum(row_hi - base, BLK)
                buf = bufs[b]

                def row_body(r, acc):
                    return tuple(
                        jnp.maximum(acc[j], buf[r, pl.ds(j * L, L)])
                        for j in range(NVEC))

                return lax.fori_loop(lo_r, hi_r, row_body, acc)

            def pair_body(p, acc, nblk=nblk):
                i0 = 2 * p
                i1 = i0 + 1

                @pl.when(i1 < nblk)
                def _():
                    start_dma(i1, 1)

                wait_dma(0)
                acc = process(i0, 0, acc)

                @pl.when(i1 + 1 < nblk)
                def _():
                    start_dma(i1 + 1, 0)

                @pl.when(i1 < nblk)
                def _():
                    wait_dma(1)

                acc = process(i1, 1, acc)
                return acc

            neg_inf = jnp.full((L,), -jnp.inf, dtype=jnp.float32)
            acc0 = tuple(neg_inf for _ in range(NVEC))
            npairs = (nblk + 1) // 2
            acc = lax.fori_loop(0, npairs, pair_body, acc0)

            for j in range(NVEC):
                res_v[pl.ds(si * D + j * L, L)] = acc[j]

        pltpu.async_copy(
            res_v, out_hbm.at[pl.ds(wid * SEGS_PER_W * D, SEGS_PER_W * D)],
            sem_out).wait()

    return k(x, starts)


def kernel(x, batch):
    seg_ids = jnp.arange(NUM_SEGMENTS + 1, dtype=batch.dtype)
    starts = jnp.searchsorted(
        batch, seg_ids, side="left", method="compare_all").astype(jnp.int32)
    starts = jnp.pad(starts, (0, STARTS_PAD - NUM_SEGMENTS - 1))
    return _seg_max_sc(x, starts).reshape(NUM_SEGMENTS, D)
